# baseline (speedup 1.0000x reference)
import functools

import jax
import jax.numpy as jnp
from jax import lax
from jax.experimental import pallas as pl
from jax.experimental.pallas import tpu as pltpu
from jax.experimental.pallas import tpu_sc as plsc

N = 20000
NV = N // 16
NLBL = 200
NCORES = 2
NSUB = 16
NW = NCORES * NSUB
LPW = 7
ML = NW * LPW
MLV = ML // 16
CAP = 224
K = 200
BIG = 1 << 30
NSLOT = NW * 8
NSV = NSLOT // 16

def _wid():
    return lax.axis_index("s") * NCORES + lax.axis_index("c")


def _k1_body(smin_hbm, smax_hbm, lmin_hbm, lmax_hbm, lists_hbm,
             smin_v, smax_v, lmin_v, lmax_v, list_v):
    w = _wid()
    base_l = w * LPW
    pltpu.sync_copy(smin_hbm, smin_v)
    pltpu.sync_copy(smax_hbm, smax_v)
    pltpu.sync_copy(lmin_hbm.at[pl.ds(base_l * 16, LPW * 16)], lmin_v)
    pltpu.sync_copy(lmax_hbm.at[pl.ds(base_l * 16, LPW * 16)], lmax_v)
    lanes = lax.iota(jnp.int32, 16)

    bigv = jnp.full((16,), BIG, jnp.int32)

    def initb(i, carry):
        list_v[pl.ds(i * 16, 16)] = bigv
        return carry

    lax.fori_loop(0, LPW * CAP // 16, initb, 0)
    bmin = [lmin_v[pl.ds(j * 16, 16)] for j in range(LPW)]
    bmax = [lmax_v[pl.ds(j * 16, 16)] for j in range(LPW)]
    blen = [bmax[j] - bmin[j] for j in range(LPW)]
    kcap = [jnp.full((16,), j * CAP + K - 1, jnp.int32) for j in range(LPW)]

    def body(i, offs):
        b = i * 16
        sm = smin_v[pl.ds(b, 16)]
        sx = smax_v[pl.ds(b, 16)]
        la = sx - sm
        iv = lanes + b
        new_offs = []
        for j in range(LPW):
            raw = jnp.minimum(sx, bmax[j]) - jnp.maximum(sm, bmin[j])
            m = (raw * 3.0) > (la + blen[j])
            off = offs[j]
            mstore = jnp.logical_and(m, off < kcap[j])
            mi = mstore.astype(jnp.int32)
            dest = jnp.minimum(off, kcap[j]) + plsc.cumsum(mi)
            plsc.store_scatter(list_v, [dest], iv, mask=mstore)
            new_offs.append(off + plsc.all_reduce_population_count(m))
        return tuple(new_offs)

    plsc.parallel_loop(
        0, NV, unroll=4,
        carry=tuple(jnp.full((16,), j * CAP - 1, jnp.int32)
                    for j in range(LPW)))(body)
    pltpu.sync_copy(list_v, lists_hbm.at[pl.ds(base_l * CAP, LPW * CAP)])


def _k2_body(lists_hbm, scores_hbm, chosen_hbm, svals_hbm,
             lists_v, scores_v, taken_v, chosen_v, svals_v):
    w = _wid()

    @pl.when(w == 0)
    def _():
        pltpu.sync_copy(lists_hbm, lists_v)
        pltpu.sync_copy(scores_hbm, scores_v)
        lanes = lax.iota(jnp.int32, 16)
        zeros = jnp.zeros((16,), jnp.int32)
        ones = jnp.ones((16,), jnp.int32)
        nvec = jnp.full((16,), N, jnp.int32)

        def zbody(i, carry):
            taken_v[pl.ds(i * 16, 16)] = zeros
            return carry

        lax.fori_loop(0, NV + 1, zbody, 0)

        lane0 = lanes == 0
        s15 = jnp.full((16,), 15, jnp.int32)

        def lbody(c, carry):
            def scan_vreg(i):
                candv = lists_v[pl.ds(c * CAP + i * 16, 16)]
                csafe = jnp.minimum(candv, nvec)
                tk = plsc.load_gather(taken_v, [csafe])
                avail = jnp.logical_and(tk == 0, candv < nvec)
                f = plsc.all_reduce_ffs(avail)
                chn = candv.at[jnp.minimum(f, s15)].get(
                    mode="promise_in_bounds")
                return jnp.where(f < 16, chn, -1), candv

            chv, candv0 = scan_vreg(jnp.int32(0))
            last_real = candv0.at[s15].get(mode="promise_in_bounds") < nvec
            pred = jnp.any(jnp.logical_and(chv < 0, last_real))

            def tail():
                def body(i, ch2):
                    chn, _ = scan_vreg(i)
                    return jnp.where(ch2 >= 0, ch2, chn)
                return lax.fori_loop(1, (K + 15) // 16, body,
                                     jnp.full((16,), -1, jnp.int32))

            chv = lax.cond(pred, tail, lambda: chv)
            has = chv >= 0
            csafe = jnp.maximum(chv, 0)
            plsc.store_scatter(taken_v, [csafe], ones,
                               mask=jnp.logical_and(lane0, has))
            sv = plsc.load_gather(scores_v, [csafe])
            chosen_v[pl.ds(c * 16, 16)] = chv
            svals_v[pl.ds(c * 16, 16)] = jnp.where(has, sv, 0.0)
            return carry

        lax.fori_loop(0, ML, lbody, 0)
        pltpu.sync_copy(chosen_v, chosen_hbm)
        pltpu.sync_copy(svals_v, svals_hbm)


def _k3_body(scores_hbm, chosen_hbm, svals_hbm, ranks_hbm,
             scores_v, ch_v, sv_v, rk_v):
    w = _wid()
    pltpu.sync_copy(scores_hbm, scores_v)
    pltpu.sync_copy(chosen_hbm.at[pl.ds(w * LPW * 16, LPW * 16)], ch_v)
    pltpu.sync_copy(svals_hbm.at[pl.ds(w * LPW * 16, LPW * 16)], sv_v)
    lanes = lax.iota(jnp.int32, 16)
    jv = [ch_v[pl.ds(j * 16, 16)] for j in range(LPW)]
    sv = [sv_v[pl.ds(j * 16, 16)] for j in range(LPW)]

    def body(i, accs):
        b = i * 16
        sc = scores_v[pl.ds(b, 16)]
        idxv = lanes + b
        return tuple(
            accs[j] + jnp.logical_or(
                sc > sv[j],
                jnp.logical_and(sc == sv[j], idxv < jv[j])
            ).astype(jnp.int32)
            for j in range(LPW))

    accs = plsc.parallel_loop(
        0, NV, unroll=4,
        carry=tuple(jnp.zeros((16,), jnp.int32) for _ in range(LPW)))(body)
    lane0 = lanes == 0
    for j in range(LPW + 1):
        if j < LPW:
            rank = jnp.where(jnp.max(jv[j]) >= 0, jnp.sum(accs[j]), BIG)
        else:
            rank = jnp.int32(BIG)
        plsc.store_scatter(rk_v, [jnp.full((16,), j, jnp.int32)],
                           jnp.full((16,), rank, jnp.int32), mask=lane0)
    pltpu.sync_copy(rk_v, ranks_hbm.at[pl.ds(w * 8, 8)])


def _k4_body(ranks_hbm, ap_hbm, rk_v, p_v, out_v):
    w = _wid()

    @pl.when(w == 0)
    def _():
        pltpu.sync_copy(ranks_hbm, rk_v)
        lanes = lax.iota(jnp.int32, 16)
        lane0 = lanes == 0

        def pa(c, carry):
            cv = jnp.full((16,), c, jnp.int32)
            rcv = plsc.load_gather(rk_v, [cv])
            rc = jnp.max(rcv)

            def ib(i, acc):
                rv = rk_v[pl.ds(i * 16, 16)]
                return acc + (rv < rcv).astype(jnp.int32)

            acc = lax.fori_loop(0, NSV, ib, jnp.zeros((16,), jnp.int32))
            pos = jnp.sum(acc)
            valid = rc < BIG
            pvec = (jnp.full((16,), pos, jnp.float32) + 1.0) / \
                   (rcv.astype(jnp.float32) + 1.0)
            pvec = jnp.where(valid, pvec, -1.0)
            plsc.store_scatter(p_v, [cv], pvec, mask=lane0)
            return carry

        lax.fori_loop(0, NSLOT, pa, 0)

        def pb(c, ap):
            cv = jnp.full((16,), c, jnp.int32)
            rcv = plsc.load_gather(rk_v, [cv])
            rc = jnp.max(rcv)

            def ib(i, pm):
                rv = rk_v[pl.ds(i * 16, 16)]
                pv = p_v[pl.ds(i * 16, 16)]
                return jnp.maximum(pm, jnp.where(rv >= rcv, pv, -1.0))

            pmv = lax.fori_loop(0, NSV, ib, jnp.full((16,), -1.0, jnp.float32))
            pm = jnp.max(pmv)
            use = jnp.logical_and(rc < BIG, rc >= 1)
            return ap + jnp.where(use, pm, 0.0)

        ap = lax.fori_loop(0, NSLOT, pb, jnp.float32(0.0))
        out_v[...] = jnp.full((16,), ap * jnp.float32(1.0 / NLBL),
                              jnp.float32)
        pltpu.sync_copy(out_v, ap_hbm)


@functools.lru_cache(maxsize=1)
def _build():
    mesh = plsc.VectorSubcoreMesh(
        core_axis_name="c", subcore_axis_name="s",
        num_cores=NCORES, num_subcores=NSUB)
    k1 = pl.kernel(
        _k1_body, mesh=mesh,
        compiler_params=pltpu.CompilerParams(needs_layout_passes=False),
        out_type=jax.ShapeDtypeStruct((ML * CAP,), jnp.int32),
        scratch_types=[pltpu.VMEM((N,), jnp.float32),
                       pltpu.VMEM((N,), jnp.float32),
                       pltpu.VMEM((LPW * 16,), jnp.float32),
                       pltpu.VMEM((LPW * 16,), jnp.float32),
                       pltpu.VMEM((LPW * CAP,), jnp.int32)])
    k2 = pl.kernel(
        _k2_body, mesh=mesh,
        compiler_params=pltpu.CompilerParams(needs_layout_passes=False),
        out_type=[jax.ShapeDtypeStruct((ML * 16,), jnp.int32),
                  jax.ShapeDtypeStruct((ML * 16,), jnp.float32)],
        scratch_types=[pltpu.VMEM((ML * CAP,), jnp.int32),
                       pltpu.VMEM((N,), jnp.float32),
                       pltpu.VMEM((N + 16,), jnp.int32),
                       pltpu.VMEM((ML * 16,), jnp.int32),
                       pltpu.VMEM((ML * 16,), jnp.float32)])
    k3 = pl.kernel(
        _k3_body, mesh=mesh,
        compiler_params=pltpu.CompilerParams(needs_layout_passes=False),
        out_type=jax.ShapeDtypeStruct((NSLOT,), jnp.int32),
        scratch_types=[pltpu.VMEM((N,), jnp.float32),
                       pltpu.VMEM((LPW * 16,), jnp.int32),
                       pltpu.VMEM((LPW * 16,), jnp.float32),
                       pltpu.VMEM((8,), jnp.int32)])
    k4 = pl.kernel(
        _k4_body, mesh=mesh,
        compiler_params=pltpu.CompilerParams(needs_layout_passes=False),
        out_type=jax.ShapeDtypeStruct((16,), jnp.float32),
        scratch_types=[pltpu.VMEM((NSLOT,), jnp.int32),
                       pltpu.VMEM((NSLOT,), jnp.float32),
                       pltpu.VMEM((16,), jnp.float32)])
    return k1, k2, k3, k4


def kernel(scores, segments, labels):
    smin = segments[:, 0]
    smax = segments[:, 1]
    lmin = jnp.concatenate(
        [labels[:, 0], jnp.full((ML - NLBL,), -2.0, jnp.float32)])
    lmax = jnp.concatenate(
        [labels[:, 1], jnp.full((ML - NLBL,), -1.0, jnp.float32)])
    lmin16 = jnp.broadcast_to(lmin[:, None], (ML, 16)).reshape(ML * 16)
    lmax16 = jnp.broadcast_to(lmax[:, None], (ML, 16)).reshape(ML * 16)

    k1, k2, k3, k4 = _build()
    lists = k1(smin, smax, lmin16, lmax16)
    chosen, svals = k2(lists, scores)
    ranks = k3(scores, chosen, svals)
    ap16 = k4(ranks)
    return ap16[0]


# speedup vs baseline: 1.3099x; 1.3099x over previous
import functools

import jax
import jax.numpy as jnp
from jax import lax
from jax.experimental import pallas as pl
from jax.experimental.pallas import tpu as pltpu
from jax.experimental.pallas import tpu_sc as plsc

N = 20000
NV = N // 16
NLBL = 200
NCORES = 2
NSUB = 16
NW = NCORES * NSUB
LPW = 7
ML = NW * LPW
MLV = ML // 16
CAP = 224
K = 200
BIG = 1 << 30
NSLOT = NW * 8
NSV = NSLOT // 16

def _wid():
    return lax.axis_index("s") * NCORES + lax.axis_index("c")


def _k1_body(smin_hbm, smax_hbm, lmin_hbm, lmax_hbm, lists_hbm,
             smin_v, smax_v, lmin_v, lmax_v, list_v):
    w = _wid()
    base_l = w * LPW
    pltpu.sync_copy(smin_hbm, smin_v)
    pltpu.sync_copy(smax_hbm, smax_v)
    pltpu.sync_copy(lmin_hbm.at[pl.ds(base_l * 16, LPW * 16)], lmin_v)
    pltpu.sync_copy(lmax_hbm.at[pl.ds(base_l * 16, LPW * 16)], lmax_v)
    lanes = lax.iota(jnp.int32, 16)

    bigv = jnp.full((16,), BIG, jnp.int32)

    def initb(i, carry):
        list_v[pl.ds(i * 16, 16)] = bigv
        return carry

    lax.fori_loop(0, LPW * CAP // 16, initb, 0)
    bmin = [lmin_v[pl.ds(j * 16, 16)] for j in range(LPW)]
    bmax = [lmax_v[pl.ds(j * 16, 16)] for j in range(LPW)]
    blen = [bmax[j] - bmin[j] for j in range(LPW)]
    kcap = [jnp.full((16,), j * CAP + K - 1, jnp.int32) for j in range(LPW)]

    def body(i, offs):
        for half in range(2):
            b = i * 32 + half * 16
            sm = smin_v[pl.ds(b, 16)]
            sx = smax_v[pl.ds(b, 16)]
            la = sx - sm
            iv = lanes + b
            new_offs = []
            for j in range(LPW):
                raw = jnp.minimum(sx, bmax[j]) - jnp.maximum(sm, bmin[j])
                m = (raw * 3.0) > (la + blen[j])
                off = offs[j]
                mstore = jnp.logical_and(m, off < kcap[j])
                mi = mstore.astype(jnp.int32)
                dest = jnp.minimum(off, kcap[j]) + plsc.cumsum(mi)
                plsc.store_scatter(list_v, [dest], iv, mask=mstore)
                new_offs.append(off + plsc.all_reduce_population_count(m))
            offs = tuple(new_offs)
        return offs

    lax.fori_loop(
        0, NV // 2, body,
        tuple(jnp.full((16,), j * CAP - 1, jnp.int32) for j in range(LPW)))
    pltpu.sync_copy(list_v, lists_hbm.at[pl.ds(base_l * CAP, LPW * CAP)])


def _k2_body(lists_hbm, scores_hbm, chosen_hbm, svals_hbm,
             lists_v, scores_v, taken_v, chosen_v, svals_v):
    w = _wid()

    @pl.when(w == 0)
    def _():
        pltpu.sync_copy(lists_hbm, lists_v)
        pltpu.sync_copy(scores_hbm, scores_v)
        lanes = lax.iota(jnp.int32, 16)
        zeros = jnp.zeros((16,), jnp.int32)
        ones = jnp.ones((16,), jnp.int32)
        nvec = jnp.full((16,), N, jnp.int32)

        def zbody(i, carry):
            taken_v[pl.ds(i * 16, 16)] = zeros
            return carry

        lax.fori_loop(0, NV + 1, zbody, 0)

        lane0 = lanes == 0
        s15 = jnp.full((16,), 15, jnp.int32)

        def lbody(c, carry):
            def scan_vreg(i):
                candv = lists_v[pl.ds(c * CAP + i * 16, 16)]
                csafe = jnp.minimum(candv, nvec)
                tk = plsc.load_gather(taken_v, [csafe])
                avail = jnp.logical_and(tk == 0, candv < nvec)
                f = plsc.all_reduce_ffs(avail)
                chn = candv.at[jnp.minimum(f, s15)].get(
                    mode="promise_in_bounds")
                return jnp.where(f < 16, chn, -1), candv

            chv, candv0 = scan_vreg(jnp.int32(0))
            last_real = candv0.at[s15].get(mode="promise_in_bounds") < nvec
            pred = jnp.any(jnp.logical_and(chv < 0, last_real))

            def tail():
                def body(i, ch2):
                    chn, _ = scan_vreg(i)
                    return jnp.where(ch2 >= 0, ch2, chn)
                return lax.fori_loop(1, (K + 15) // 16, body,
                                     jnp.full((16,), -1, jnp.int32))

            chv = lax.cond(pred, tail, lambda: chv)
            has = chv >= 0
            csafe = jnp.maximum(chv, 0)
            plsc.store_scatter(taken_v, [csafe], ones,
                               mask=jnp.logical_and(lane0, has))
            sv = plsc.load_gather(scores_v, [csafe])
            chosen_v[pl.ds(c * 16, 16)] = chv
            svals_v[pl.ds(c * 16, 16)] = jnp.where(has, sv, 0.0)
            return carry

        lax.fori_loop(0, ML, lbody, 0)
        pltpu.sync_copy(chosen_v, chosen_hbm)
        pltpu.sync_copy(svals_v, svals_hbm)


def _k3_body(scores_hbm, chosen_hbm, svals_hbm, ranks_hbm,
             scores_v, ch_v, sv_v, rk_v):
    w = _wid()
    pltpu.sync_copy(scores_hbm, scores_v)
    pltpu.sync_copy(chosen_hbm.at[pl.ds(w * LPW * 16, LPW * 16)], ch_v)
    pltpu.sync_copy(svals_hbm.at[pl.ds(w * LPW * 16, LPW * 16)], sv_v)
    lanes = lax.iota(jnp.int32, 16)
    jv = [ch_v[pl.ds(j * 16, 16)] for j in range(LPW)]
    sv = [sv_v[pl.ds(j * 16, 16)] for j in range(LPW)]

    def body(i, accs):
        for half in range(2):
            b = i * 32 + half * 16
            sc = scores_v[pl.ds(b, 16)]
            idxv = lanes + b
            accs = tuple(
                accs[j] + jnp.logical_or(
                    sc > sv[j],
                    jnp.logical_and(sc == sv[j], idxv < jv[j])
                ).astype(jnp.int32)
                for j in range(LPW))
        return accs

    accs = lax.fori_loop(
        0, NV // 2, body,
        tuple(jnp.zeros((16,), jnp.int32) for _ in range(LPW)))
    lane0 = lanes == 0
    for j in range(LPW + 1):
        if j < LPW:
            rank = jnp.where(jnp.max(jv[j]) >= 0, jnp.sum(accs[j]), BIG)
        else:
            rank = jnp.int32(BIG)
        plsc.store_scatter(rk_v, [jnp.full((16,), j, jnp.int32)],
                           jnp.full((16,), rank, jnp.int32), mask=lane0)
    pltpu.sync_copy(rk_v, ranks_hbm.at[pl.ds(w * 8, 8)])


def _k4_body(ranks_hbm, ap_hbm, rk_v, p_v, out_v):
    w = _wid()

    @pl.when(w == 0)
    def _():
        pltpu.sync_copy(ranks_hbm, rk_v)
        lanes = lax.iota(jnp.int32, 16)
        lane0 = lanes == 0

        def pa(c, carry):
            cv = jnp.full((16,), c, jnp.int32)
            rcv = plsc.load_gather(rk_v, [cv])
            rc = jnp.max(rcv)

            def ib(i, acc):
                rv = rk_v[pl.ds(i * 16, 16)]
                return acc + (rv < rcv).astype(jnp.int32)

            acc = lax.fori_loop(0, NSV, ib, jnp.zeros((16,), jnp.int32))
            pos = jnp.sum(acc)
            valid = rc < BIG
            pvec = (jnp.full((16,), pos, jnp.float32) + 1.0) / \
                   (rcv.astype(jnp.float32) + 1.0)
            pvec = jnp.where(valid, pvec, -1.0)
            plsc.store_scatter(p_v, [cv], pvec, mask=lane0)
            return carry

        lax.fori_loop(0, NSLOT, pa, 0)

        def pb(c, ap):
            cv = jnp.full((16,), c, jnp.int32)
            rcv = plsc.load_gather(rk_v, [cv])
            rc = jnp.max(rcv)

            def ib(i, pm):
                rv = rk_v[pl.ds(i * 16, 16)]
                pv = p_v[pl.ds(i * 16, 16)]
                return jnp.maximum(pm, jnp.where(rv >= rcv, pv, -1.0))

            pmv = lax.fori_loop(0, NSV, ib, jnp.full((16,), -1.0, jnp.float32))
            pm = jnp.max(pmv)
            use = jnp.logical_and(rc < BIG, rc >= 1)
            return ap + jnp.where(use, pm, 0.0)

        ap = lax.fori_loop(0, NSLOT, pb, jnp.float32(0.0))
        out_v[...] = jnp.full((16,), ap * jnp.float32(1.0 / NLBL),
                              jnp.float32)
        pltpu.sync_copy(out_v, ap_hbm)


@functools.lru_cache(maxsize=1)
def _build():
    mesh = plsc.VectorSubcoreMesh(
        core_axis_name="c", subcore_axis_name="s",
        num_cores=NCORES, num_subcores=NSUB)
    k1 = pl.kernel(
        _k1_body, mesh=mesh,
        compiler_params=pltpu.CompilerParams(needs_layout_passes=False),
        out_type=jax.ShapeDtypeStruct((ML * CAP,), jnp.int32),
        scratch_types=[pltpu.VMEM((N,), jnp.float32),
                       pltpu.VMEM((N,), jnp.float32),
                       pltpu.VMEM((LPW * 16,), jnp.float32),
                       pltpu.VMEM((LPW * 16,), jnp.float32),
                       pltpu.VMEM((LPW * CAP,), jnp.int32)])
    k2 = pl.kernel(
        _k2_body, mesh=mesh,
        compiler_params=pltpu.CompilerParams(needs_layout_passes=False),
        out_type=[jax.ShapeDtypeStruct((ML * 16,), jnp.int32),
                  jax.ShapeDtypeStruct((ML * 16,), jnp.float32)],
        scratch_types=[pltpu.VMEM((ML * CAP,), jnp.int32),
                       pltpu.VMEM((N,), jnp.float32),
                       pltpu.VMEM((N + 16,), jnp.int32),
                       pltpu.VMEM((ML * 16,), jnp.int32),
                       pltpu.VMEM((ML * 16,), jnp.float32)])
    k3 = pl.kernel(
        _k3_body, mesh=mesh,
        compiler_params=pltpu.CompilerParams(needs_layout_passes=False),
        out_type=jax.ShapeDtypeStruct((NSLOT,), jnp.int32),
        scratch_types=[pltpu.VMEM((N,), jnp.float32),
                       pltpu.VMEM((LPW * 16,), jnp.int32),
                       pltpu.VMEM((LPW * 16,), jnp.float32),
                       pltpu.VMEM((8,), jnp.int32)])
    k4 = pl.kernel(
        _k4_body, mesh=mesh,
        compiler_params=pltpu.CompilerParams(needs_layout_passes=False),
        out_type=jax.ShapeDtypeStruct((16,), jnp.float32),
        scratch_types=[pltpu.VMEM((NSLOT,), jnp.int32),
                       pltpu.VMEM((NSLOT,), jnp.float32),
                       pltpu.VMEM((16,), jnp.float32)])
    return k1, k2, k3, k4


def kernel(scores, segments, labels):
    smin = segments[:, 0]
    smax = segments[:, 1]
    lmin = jnp.concatenate(
        [labels[:, 0], jnp.full((ML - NLBL,), -2.0, jnp.float32)])
    lmax = jnp.concatenate(
        [labels[:, 1], jnp.full((ML - NLBL,), -1.0, jnp.float32)])
    lmin16 = jnp.broadcast_to(lmin[:, None], (ML, 16)).reshape(ML * 16)
    lmax16 = jnp.broadcast_to(lmax[:, None], (ML, 16)).reshape(ML * 16)

    k1, k2, k3, k4 = _build()
    lists = k1(smin, smax, lmin16, lmax16)
    chosen, svals = k2(lists, scores)
    ranks = k3(scores, chosen, svals)
    ap16 = k4(ranks)
    return ap16[0]


# speedup vs baseline: 2.0103x; 1.5347x over previous
import functools

import jax
import jax.numpy as jnp
from jax import lax
from jax.experimental import pallas as pl
from jax.experimental.pallas import tpu as pltpu
from jax.experimental.pallas import tpu_sc as plsc

N = 20000
NV = N // 16
NLBL = 200
NCORES = 2
NSUB = 16
NW = NCORES * NSUB
LPW = 7
ML = NW * LPW
MLV = ML // 16
CAP = 224
K = 200
BIG = 1 << 30
NSLOT = NW * 8
NSV = NSLOT // 16

def _wid():
    return lax.axis_index("s") * NCORES + lax.axis_index("c")


def _k1_body(smin_hbm, smax_hbm, lmin_hbm, lmax_hbm, lists_hbm,
             smin_v, smax_v, lmin_v, lmax_v, list_v):
    w = _wid()
    base_l = w * LPW
    pltpu.sync_copy(smin_hbm, smin_v)
    pltpu.sync_copy(smax_hbm, smax_v)
    pltpu.sync_copy(lmin_hbm.at[pl.ds(base_l * 16, LPW * 16)], lmin_v)
    pltpu.sync_copy(lmax_hbm.at[pl.ds(base_l * 16, LPW * 16)], lmax_v)
    lanes = lax.iota(jnp.int32, 16)

    bigv = jnp.full((16,), BIG, jnp.int32)

    def initb(i, carry):
        list_v[pl.ds(i * 16, 16)] = bigv
        return carry

    lax.fori_loop(0, LPW * CAP // 16, initb, 0)
    bmin = [lmin_v[pl.ds(j * 16, 16)] for j in range(LPW)]
    bmax = [lmax_v[pl.ds(j * 16, 16)] for j in range(LPW)]
    blen = [bmax[j] - bmin[j] for j in range(LPW)]
    kcap = [jnp.full((16,), j * CAP + K - 1, jnp.int32) for j in range(LPW)]

    def body(i, offs):
        b = i * 16
        sm = smin_v[pl.ds(b, 16)]
        sx = smax_v[pl.ds(b, 16)]
        la = sx - sm
        iv = lanes + b
        new_offs = []
        for j in range(LPW):
            raw = jnp.minimum(sx, bmax[j]) - jnp.maximum(sm, bmin[j])
            m = (raw * 3.0) > (la + blen[j])
            off = offs[j]
            mstore = jnp.logical_and(m, off < kcap[j])
            mi = mstore.astype(jnp.int32)
            dest = jnp.minimum(off, kcap[j]) + plsc.cumsum(mi)
            plsc.store_scatter(list_v, [dest], iv, mask=mstore)
            new_offs.append(off + plsc.all_reduce_population_count(m))
        return tuple(new_offs)

    lax.fori_loop(
        0, NV, body,
        tuple(jnp.full((16,), j * CAP - 1, jnp.int32) for j in range(LPW)))
    pltpu.sync_copy(list_v, lists_hbm.at[pl.ds(base_l * CAP, LPW * CAP)])


def _k2_body(lists_hbm, scores_hbm, chosen_hbm, svals_hbm,
             lists_v, scores_v, taken_v, chosen_v, svals_v):
    w = _wid()

    @pl.when(w == 0)
    def _():
        pltpu.sync_copy(lists_hbm, lists_v)
        pltpu.sync_copy(scores_hbm, scores_v)
        lanes = lax.iota(jnp.int32, 16)
        zeros = jnp.zeros((16,), jnp.int32)
        ones = jnp.ones((16,), jnp.int32)
        nvec = jnp.full((16,), N, jnp.int32)

        def zbody(i, carry):
            taken_v[pl.ds(i * 16, 16)] = zeros
            return carry

        lax.fori_loop(0, NV + 1, zbody, 0)

        lane0 = lanes == 0
        s15 = jnp.full((16,), 15, jnp.int32)

        def lbody(c, carry):
            def scan_vreg(i):
                candv = lists_v[pl.ds(c * CAP + i * 16, 16)]
                csafe = jnp.minimum(candv, nvec)
                tk = plsc.load_gather(taken_v, [csafe])
                avail = jnp.logical_and(tk == 0, candv < nvec)
                f = plsc.all_reduce_ffs(avail)
                chn = candv.at[jnp.minimum(f, s15)].get(
                    mode="promise_in_bounds")
                return jnp.where(f < 16, chn, -1), candv

            chv, candv0 = scan_vreg(jnp.int32(0))
            last_real = candv0.at[s15].get(mode="promise_in_bounds") < nvec
            pred = jnp.any(jnp.logical_and(chv < 0, last_real))

            def tail():
                def body(i, ch2):
                    chn, _ = scan_vreg(i)
                    return jnp.where(ch2 >= 0, ch2, chn)
                return lax.fori_loop(1, (K + 15) // 16, body,
                                     jnp.full((16,), -1, jnp.int32))

            chv = lax.cond(pred, tail, lambda: chv)
            has = chv >= 0
            csafe = jnp.maximum(chv, 0)
            plsc.store_scatter(taken_v, [csafe], ones,
                               mask=jnp.logical_and(lane0, has))
            sv = plsc.load_gather(scores_v, [csafe])
            chosen_v[pl.ds(c * 16, 16)] = chv
            svals_v[pl.ds(c * 16, 16)] = jnp.where(has, sv, 0.0)
            return carry

        lax.fori_loop(0, ML, lbody, 0)
        pltpu.sync_copy(chosen_v, chosen_hbm)
        pltpu.sync_copy(svals_v, svals_hbm)


def _k3_body(scores_hbm, chosen_hbm, svals_hbm, ranks_hbm,
             scores_v, ch_v, sv_v, rk_v):
    w = _wid()
    pltpu.sync_copy(scores_hbm, scores_v)
    pltpu.sync_copy(chosen_hbm.at[pl.ds(w * LPW * 16, LPW * 16)], ch_v)
    pltpu.sync_copy(svals_hbm.at[pl.ds(w * LPW * 16, LPW * 16)], sv_v)
    lanes = lax.iota(jnp.int32, 16)
    jv = [ch_v[pl.ds(j * 16, 16)] for j in range(LPW)]
    sv = [sv_v[pl.ds(j * 16, 16)] for j in range(LPW)]

    def body(i, accs):
        b = i * 16
        sc = scores_v[pl.ds(b, 16)]
        idxv = lanes + b
        return tuple(
            accs[j] + jnp.logical_or(
                sc > sv[j],
                jnp.logical_and(sc == sv[j], idxv < jv[j])
            ).astype(jnp.int32)
            for j in range(LPW))

    accs = lax.fori_loop(
        0, NV, body, tuple(jnp.zeros((16,), jnp.int32) for _ in range(LPW)))
    lane0 = lanes == 0
    for j in range(LPW + 1):
        if j < LPW:
            rank = jnp.where(jnp.max(jv[j]) >= 0, jnp.sum(accs[j]), BIG)
        else:
            rank = jnp.int32(BIG)
        plsc.store_scatter(rk_v, [jnp.full((16,), j, jnp.int32)],
                           jnp.full((16,), rank, jnp.int32), mask=lane0)
    pltpu.sync_copy(rk_v, ranks_hbm.at[pl.ds(w * 8, 8)])


def _k4_body(ranks_hbm, ap_hbm, rk_v, p_v, out_v):
    w = _wid()

    @pl.when(w == 0)
    def _():
        pltpu.sync_copy(ranks_hbm, rk_v)
        lanes = lax.iota(jnp.int32, 16)
        lane0 = lanes == 0

        neg1 = jnp.full((16,), -1.0, jnp.float32)

        def spi(i, carry):
            p_v[pl.ds(i * 16, 16)] = neg1
            return carry

        lax.fori_loop(0, NSV, spi, 0)

        def pa(c, carry):
            cv = jnp.full((16,), c, jnp.int32)
            rcv = plsc.load_gather(rk_v, [cv])

            def ib(i, acc):
                rv = rk_v[pl.ds(i * 16, 16)]
                return acc + (rv < rcv).astype(jnp.int32)

            acc = lax.fori_loop(0, NSV, ib, jnp.zeros((16,), jnp.int32))
            pos = jnp.sum(acc)
            validv = rcv < BIG
            pvec = (jnp.full((16,), pos, jnp.float32) + 1.0) / \
                   (rcv.astype(jnp.float32) + 1.0)
            posv = jnp.full((16,), pos, jnp.int32)
            plsc.store_scatter(p_v, [posv], pvec,
                               mask=jnp.logical_and(lane0, validv))
            return carry

        lax.fori_loop(0, NSLOT, pa, 0)

        def r0b(i, acc):
            rv = rk_v[pl.ds(i * 16, 16)]
            return jnp.logical_or(acc, rv == 0)

        r0v = lax.fori_loop(0, NSV, r0b, lanes != lanes)
        has_r0 = jnp.any(r0v)

        def pb(t, carry):
            runmax, ap = carry
            i = NSV - 1 - t
            v = p_v[pl.ds(i * 16, 16)]
            suf = lax.rev(plsc.cummax(lax.rev(v, dimensions=(0,))),
                          dimensions=(0,))
            tot = jnp.maximum(suf, runmax)
            ap = ap + jnp.sum(jnp.where(v >= 0.0, tot, 0.0))
            return jnp.full((16,), jnp.max(tot), jnp.float32), ap

        runmax, ap = lax.fori_loop(0, NSV, pb, (neg1, jnp.float32(0.0)))
        ap = ap - jnp.where(has_r0, jnp.max(runmax), 0.0)
        out_v[...] = jnp.full((16,), ap * jnp.float32(1.0 / NLBL),
                              jnp.float32)
        pltpu.sync_copy(out_v, ap_hbm)


@functools.lru_cache(maxsize=1)
def _build():
    mesh = plsc.VectorSubcoreMesh(
        core_axis_name="c", subcore_axis_name="s",
        num_cores=NCORES, num_subcores=NSUB)
    k1 = pl.kernel(
        _k1_body, mesh=mesh,
        compiler_params=pltpu.CompilerParams(needs_layout_passes=False),
        out_type=jax.ShapeDtypeStruct((ML * CAP,), jnp.int32),
        scratch_types=[pltpu.VMEM((N,), jnp.float32),
                       pltpu.VMEM((N,), jnp.float32),
                       pltpu.VMEM((LPW * 16,), jnp.float32),
                       pltpu.VMEM((LPW * 16,), jnp.float32),
                       pltpu.VMEM((LPW * CAP,), jnp.int32)])
    k2 = pl.kernel(
        _k2_body, mesh=mesh,
        compiler_params=pltpu.CompilerParams(needs_layout_passes=False),
        out_type=[jax.ShapeDtypeStruct((ML * 16,), jnp.int32),
                  jax.ShapeDtypeStruct((ML * 16,), jnp.float32)],
        scratch_types=[pltpu.VMEM((ML * CAP,), jnp.int32),
                       pltpu.VMEM((N,), jnp.float32),
                       pltpu.VMEM((N + 16,), jnp.int32),
                       pltpu.VMEM((ML * 16,), jnp.int32),
                       pltpu.VMEM((ML * 16,), jnp.float32)])
    k3 = pl.kernel(
        _k3_body, mesh=mesh,
        compiler_params=pltpu.CompilerParams(needs_layout_passes=False),
        out_type=jax.ShapeDtypeStruct((NSLOT,), jnp.int32),
        scratch_types=[pltpu.VMEM((N,), jnp.float32),
                       pltpu.VMEM((LPW * 16,), jnp.int32),
                       pltpu.VMEM((LPW * 16,), jnp.float32),
                       pltpu.VMEM((8,), jnp.int32)])
    k4 = pl.kernel(
        _k4_body, mesh=mesh,
        compiler_params=pltpu.CompilerParams(needs_layout_passes=False),
        out_type=jax.ShapeDtypeStruct((16,), jnp.float32),
        scratch_types=[pltpu.VMEM((NSLOT,), jnp.int32),
                       pltpu.VMEM((NSLOT,), jnp.float32),
                       pltpu.VMEM((16,), jnp.float32)])
    return k1, k2, k3, k4


def kernel(scores, segments, labels):
    smin = segments[:, 0]
    smax = segments[:, 1]
    lmin = jnp.concatenate(
        [labels[:, 0], jnp.full((ML - NLBL,), -2.0, jnp.float32)])
    lmax = jnp.concatenate(
        [labels[:, 1], jnp.full((ML - NLBL,), -1.0, jnp.float32)])
    lmin16 = jnp.broadcast_to(lmin[:, None], (ML, 16)).reshape(ML * 16)
    lmax16 = jnp.broadcast_to(lmax[:, None], (ML, 16)).reshape(ML * 16)

    k1, k2, k3, k4 = _build()
    lists = k1(smin, smax, lmin16, lmax16)
    chosen, svals = k2(lists, scores)
    ranks = k3(scores, chosen, svals)
    ap16 = k4(ranks)
    return ap16[0]


# speedup vs baseline: 2.0478x; 1.0187x over previous
import functools

import jax
import jax.numpy as jnp
from jax import lax
from jax.experimental import pallas as pl
from jax.experimental.pallas import tpu as pltpu
from jax.experimental.pallas import tpu_sc as plsc

N = 20000
NV = N // 16
NLBL = 200
NCORES = 2
NSUB = 16
NW = NCORES * NSUB
LPW = 7
ML = NW * LPW
MLV = ML // 16
CAP = 224
K = 200
BIG = 1 << 30
NSLOT = NW * 8
NSV = NSLOT // 16

def _wid():
    return lax.axis_index("s") * NCORES + lax.axis_index("c")


def _k1_body(smin_hbm, smax_hbm, lmin_hbm, lmax_hbm, lists_hbm,
             smin_v, smax_v, lmin_v, lmax_v, list_v):
    w = _wid()
    base_l = w * LPW
    pltpu.sync_copy(smin_hbm, smin_v)
    pltpu.sync_copy(smax_hbm, smax_v)
    pltpu.sync_copy(lmin_hbm.at[pl.ds(base_l * 16, LPW * 16)], lmin_v)
    pltpu.sync_copy(lmax_hbm.at[pl.ds(base_l * 16, LPW * 16)], lmax_v)
    lanes = lax.iota(jnp.int32, 16)

    bigv = jnp.full((16,), BIG, jnp.int32)

    def initb(i, carry):
        list_v[pl.ds(i * 16, 16)] = bigv
        return carry

    lax.fori_loop(0, LPW * CAP // 16, initb, 0)
    bmin = [lmin_v[pl.ds(j * 16, 16)] for j in range(LPW)]
    bmax = [lmax_v[pl.ds(j * 16, 16)] for j in range(LPW)]
    blen = [bmax[j] - bmin[j] for j in range(LPW)]
    kcap = [jnp.full((16,), j * CAP + K - 1, jnp.int32) for j in range(LPW)]

    def body(i, offs):
        b = i * 16
        sm = smin_v[pl.ds(b, 16)]
        sx = smax_v[pl.ds(b, 16)]
        la = sx - sm
        iv = lanes + b
        new_offs = []
        for j in range(LPW):
            raw = jnp.minimum(sx, bmax[j]) - jnp.maximum(sm, bmin[j])
            m = (raw * 3.0) > (la + blen[j])
            off = offs[j]
            mstore = jnp.logical_and(m, off < kcap[j])
            mi = mstore.astype(jnp.int32)
            dest = jnp.minimum(off, kcap[j]) + plsc.cumsum(mi)
            plsc.store_scatter(list_v, [dest], iv, mask=mstore)
            new_offs.append(off + plsc.all_reduce_population_count(m))
        return tuple(new_offs)

    lax.fori_loop(
        0, NV, body,
        tuple(jnp.full((16,), j * CAP - 1, jnp.int32) for j in range(LPW)))
    pltpu.sync_copy(list_v, lists_hbm.at[pl.ds(base_l * CAP, LPW * CAP)])


def _k2_body(lists_hbm, scores_hbm, chosen_hbm, svals_hbm,
             lists_v, scores_v, taken_v, chosen_v, svals_v):
    w = _wid()

    @pl.when(w == 0)
    def _():
        pltpu.sync_copy(lists_hbm, lists_v)
        pltpu.sync_copy(scores_hbm, scores_v)
        lanes = lax.iota(jnp.int32, 16)
        zeros = jnp.zeros((16,), jnp.int32)
        ones = jnp.ones((16,), jnp.int32)
        nvec = jnp.full((16,), N, jnp.int32)

        def zbody(i, carry):
            taken_v[pl.ds(i * 16, 16)] = zeros
            return carry

        lax.fori_loop(0, NV + 1, zbody, 0)

        lane0 = lanes == 0
        s15 = jnp.full((16,), 15, jnp.int32)

        def lbody(c, carry):
            def scan_vreg(i):
                candv = lists_v[pl.ds(c * CAP + i * 16, 16)]
                csafe = jnp.minimum(candv, nvec)
                tk = plsc.load_gather(taken_v, [csafe])
                avail = jnp.logical_and(tk == 0, candv < nvec)
                f = plsc.all_reduce_ffs(avail)
                chn = candv.at[jnp.minimum(f, s15)].get(
                    mode="promise_in_bounds")
                return jnp.where(f < 16, chn, -1), candv

            chv, candv0 = scan_vreg(jnp.int32(0))
            last_real = candv0.at[s15].get(mode="promise_in_bounds") < nvec
            pred = jnp.any(jnp.logical_and(chv < 0, last_real))

            def tail():
                def body(i, ch2):
                    chn, _ = scan_vreg(i)
                    return jnp.where(ch2 >= 0, ch2, chn)
                return lax.fori_loop(1, (K + 15) // 16, body,
                                     jnp.full((16,), -1, jnp.int32))

            chv = lax.cond(pred, tail, lambda: chv)
            has = chv >= 0
            csafe = jnp.maximum(chv, 0)
            plsc.store_scatter(taken_v, [csafe], ones,
                               mask=jnp.logical_and(lane0, has))
            sv = plsc.load_gather(scores_v, [csafe])
            chosen_v[pl.ds(c * 16, 16)] = chv
            svals_v[pl.ds(c * 16, 16)] = jnp.where(has, sv, 0.0)
            return carry

        lax.fori_loop(0, ML, lbody, 0)
        pltpu.sync_copy(chosen_v, chosen_hbm)
        pltpu.sync_copy(svals_v, svals_hbm)


def _k3_body(scores_hbm, chosen_hbm, svals_hbm, ranks_hbm,
             scores_v, ch_v, sv_v, rk_v):
    w = _wid()
    pltpu.sync_copy(scores_hbm, scores_v)
    pltpu.sync_copy(chosen_hbm.at[pl.ds(w * LPW * 16, LPW * 16)], ch_v)
    pltpu.sync_copy(svals_hbm.at[pl.ds(w * LPW * 16, LPW * 16)], sv_v)
    lanes = lax.iota(jnp.int32, 16)
    jv = [ch_v[pl.ds(j * 16, 16)] for j in range(LPW)]
    sv = [sv_v[pl.ds(j * 16, 16)] for j in range(LPW)]

    def body(i, accs):
        b = i * 16
        sc = scores_v[pl.ds(b, 16)]
        idxv = lanes + b
        return tuple(
            accs[j] + jnp.logical_or(
                sc > sv[j],
                jnp.logical_and(sc == sv[j], idxv < jv[j])
            ).astype(jnp.int32)
            for j in range(LPW))

    accs = lax.fori_loop(
        0, NV, body, tuple(jnp.zeros((16,), jnp.int32) for _ in range(LPW)))
    lane0 = lanes == 0
    for j in range(LPW + 1):
        if j < LPW:
            rank = jnp.where(jnp.max(jv[j]) >= 0, jnp.sum(accs[j]), BIG)
        else:
            rank = jnp.int32(BIG)
        plsc.store_scatter(rk_v, [jnp.full((16,), j, jnp.int32)],
                           jnp.full((16,), rank, jnp.int32), mask=lane0)
    pltpu.sync_copy(rk_v, ranks_hbm.at[pl.ds(w * 8, 8)])


def _k4_body(ranks_hbm, ap_hbm, rk_v, p_v, out_v):
    w = _wid()

    @pl.when(w == 0)
    def _():
        pltpu.sync_copy(ranks_hbm, rk_v)
        lanes = lax.iota(jnp.int32, 16)
        lane0 = lanes == 0

        neg1 = jnp.full((16,), -1.0, jnp.float32)

        def spi(i, carry):
            p_v[pl.ds(i * 16, 16)] = neg1
            return carry

        lax.fori_loop(0, NSV, spi, 0)

        def pa(c, carry):
            cv = jnp.full((16,), c, jnp.int32)
            rcv = plsc.load_gather(rk_v, [cv])

            def ib(i, acc):
                rv = rk_v[pl.ds(i * 16, 16)]
                return acc + (rv < rcv).astype(jnp.int32)

            acc = lax.fori_loop(0, NSV, ib, jnp.zeros((16,), jnp.int32))
            pos = jnp.sum(acc)
            validv = rcv < BIG
            pvec = (jnp.full((16,), pos, jnp.float32) + 1.0) / \
                   (rcv.astype(jnp.float32) + 1.0)
            posv = jnp.full((16,), pos, jnp.int32)
            plsc.store_scatter(p_v, [posv], pvec,
                               mask=jnp.logical_and(lane0, validv))
            return carry

        lax.fori_loop(0, NSLOT, pa, 0)

        def r0b(i, acc):
            rv = rk_v[pl.ds(i * 16, 16)]
            return jnp.logical_or(acc, rv == 0)

        r0v = lax.fori_loop(0, NSV, r0b, lanes != lanes)
        has_r0 = jnp.any(r0v)

        def pb(t, carry):
            runmax, ap = carry
            i = NSV - 1 - t
            v = p_v[pl.ds(i * 16, 16)]
            suf = lax.rev(plsc.cummax(lax.rev(v, dimensions=(0,))),
                          dimensions=(0,))
            tot = jnp.maximum(suf, runmax)
            ap = ap + jnp.sum(jnp.where(v >= 0.0, tot, 0.0))
            return jnp.full((16,), jnp.max(tot), jnp.float32), ap

        runmax, ap = lax.fori_loop(0, NSV, pb, (neg1, jnp.float32(0.0)))
        ap = ap - jnp.where(has_r0, jnp.max(runmax), 0.0)
        out_v[...] = jnp.full((16,), ap * jnp.float32(1.0 / NLBL),
                              jnp.float32)
        pltpu.sync_copy(out_v, ap_hbm)


@functools.lru_cache(maxsize=1)
def _build():
    mesh = plsc.VectorSubcoreMesh(
        core_axis_name="c", subcore_axis_name="s",
        num_cores=NCORES, num_subcores=NSUB)
    mesh1 = plsc.VectorSubcoreMesh(
        core_axis_name="c", subcore_axis_name="s",
        num_cores=1, num_subcores=1)
    k1 = pl.kernel(
        _k1_body, mesh=mesh,
        compiler_params=pltpu.CompilerParams(needs_layout_passes=False),
        out_type=jax.ShapeDtypeStruct((ML * CAP,), jnp.int32),
        scratch_types=[pltpu.VMEM((N,), jnp.float32),
                       pltpu.VMEM((N,), jnp.float32),
                       pltpu.VMEM((LPW * 16,), jnp.float32),
                       pltpu.VMEM((LPW * 16,), jnp.float32),
                       pltpu.VMEM((LPW * CAP,), jnp.int32)])
    k2 = pl.kernel(
        _k2_body, mesh=mesh1,
        compiler_params=pltpu.CompilerParams(needs_layout_passes=False),
        out_type=[jax.ShapeDtypeStruct((ML * 16,), jnp.int32),
                  jax.ShapeDtypeStruct((ML * 16,), jnp.float32)],
        scratch_types=[pltpu.VMEM((ML * CAP,), jnp.int32),
                       pltpu.VMEM((N,), jnp.float32),
                       pltpu.VMEM((N + 16,), jnp.int32),
                       pltpu.VMEM((ML * 16,), jnp.int32),
                       pltpu.VMEM((ML * 16,), jnp.float32)])
    k3 = pl.kernel(
        _k3_body, mesh=mesh,
        compiler_params=pltpu.CompilerParams(needs_layout_passes=False),
        out_type=jax.ShapeDtypeStruct((NSLOT,), jnp.int32),
        scratch_types=[pltpu.VMEM((N,), jnp.float32),
                       pltpu.VMEM((LPW * 16,), jnp.int32),
                       pltpu.VMEM((LPW * 16,), jnp.float32),
                       pltpu.VMEM((8,), jnp.int32)])
    k4 = pl.kernel(
        _k4_body, mesh=mesh1,
        compiler_params=pltpu.CompilerParams(needs_layout_passes=False),
        out_type=jax.ShapeDtypeStruct((16,), jnp.float32),
        scratch_types=[pltpu.VMEM((NSLOT,), jnp.int32),
                       pltpu.VMEM((NSLOT,), jnp.float32),
                       pltpu.VMEM((16,), jnp.float32)])
    return k1, k2, k3, k4


def kernel(scores, segments, labels):
    smin = segments[:, 0]
    smax = segments[:, 1]
    lmin = jnp.concatenate(
        [labels[:, 0], jnp.full((ML - NLBL,), -2.0, jnp.float32)])
    lmax = jnp.concatenate(
        [labels[:, 1], jnp.full((ML - NLBL,), -1.0, jnp.float32)])
    lmin16 = jnp.broadcast_to(lmin[:, None], (ML, 16)).reshape(ML * 16)
    lmax16 = jnp.broadcast_to(lmax[:, None], (ML, 16)).reshape(ML * 16)

    k1, k2, k3, k4 = _build()
    lists = k1(smin, smax, lmin16, lmax16)
    chosen, svals = k2(lists, scores)
    ranks = k3(scores, chosen, svals)
    ap16 = k4(ranks)
    return ap16[0]


# speedup vs baseline: 2.0483x; 1.0003x over previous
import functools

import jax
import jax.numpy as jnp
from jax import lax
from jax.experimental import pallas as pl
from jax.experimental.pallas import tpu as pltpu
from jax.experimental.pallas import tpu_sc as plsc

N = 20000
NV = N // 16
NLBL = 200
NCORES = 2
NSUB = 16
NW = NCORES * NSUB
LPW = 7
ML = NW * LPW
MLV = ML // 16
CAP = 224
K = 200
BIG = 1 << 30
NSLOT = NW * 8
NSV = NSLOT // 16

def _wid():
    return lax.axis_index("s") * NCORES + lax.axis_index("c")


def _k1_body(smin_hbm, smax_hbm, lmin_hbm, lmax_hbm, lists_hbm,
             smin_v, smax_v, lmin_v, lmax_v, list_v):
    w = _wid()
    base_l = w * LPW
    pltpu.sync_copy(smin_hbm, smin_v)
    pltpu.sync_copy(smax_hbm, smax_v)
    pltpu.sync_copy(lmin_hbm.at[pl.ds(base_l * 16, LPW * 16)], lmin_v)
    pltpu.sync_copy(lmax_hbm.at[pl.ds(base_l * 16, LPW * 16)], lmax_v)
    lanes = lax.iota(jnp.int32, 16)

    bigv = jnp.full((16,), BIG, jnp.int32)

    def initb(i, carry):
        list_v[pl.ds(i * 16, 16)] = bigv
        return carry

    lax.fori_loop(0, LPW * CAP // 16, initb, 0)
    bmin = [lmin_v[pl.ds(j * 16, 16)] for j in range(LPW)]
    bmax = [lmax_v[pl.ds(j * 16, 16)] for j in range(LPW)]
    blen = [bmax[j] - bmin[j] for j in range(LPW)]
    kcap = [jnp.full((16,), j * CAP + K - 1, jnp.int32) for j in range(LPW)]

    def body(i, offs):
        b = i * 16
        sm = smin_v[pl.ds(b, 16)]
        sx = smax_v[pl.ds(b, 16)]
        la = sx - sm
        iv = lanes + b
        new_offs = []
        for j in range(LPW):
            raw = jnp.minimum(sx, bmax[j]) - jnp.maximum(sm, bmin[j])
            m = (raw * 3.0) > (la + blen[j])
            off = offs[j]
            mstore = jnp.logical_and(m, off < kcap[j])
            mi = mstore.astype(jnp.int32)
            dest = jnp.minimum(off, kcap[j]) + plsc.cumsum(mi)
            plsc.store_scatter(list_v, [dest], iv, mask=mstore)
            new_offs.append(off + plsc.all_reduce_population_count(m))
        return tuple(new_offs)

    lax.fori_loop(
        0, NV, body,
        tuple(jnp.full((16,), j * CAP - 1, jnp.int32) for j in range(LPW)))
    pltpu.sync_copy(list_v, lists_hbm.at[pl.ds(base_l * CAP, LPW * CAP)])


def _k2_body(lists_hbm, scores_hbm, chosen_hbm, svals_hbm,
             lists_v, scores_v, taken_v, chosen_v, svals_v):
    w = _wid()

    @pl.when(w == 0)
    def _():
        pltpu.sync_copy(lists_hbm, lists_v)
        pltpu.sync_copy(scores_hbm, scores_v)
        lanes = lax.iota(jnp.int32, 16)
        zeros = jnp.zeros((16,), jnp.int32)
        ones = jnp.ones((16,), jnp.int32)
        nvec = jnp.full((16,), N, jnp.int32)

        def zbody(i, carry):
            taken_v[pl.ds(i * 16, 16)] = zeros
            return carry

        lax.fori_loop(0, NV + 1, zbody, 0)

        lane0 = lanes == 0
        s15 = jnp.full((16,), 15, jnp.int32)

        def lbody(c, carry):
            def scan_vreg(i):
                candv = lists_v[pl.ds(c * CAP + i * 16, 16)]
                csafe = jnp.minimum(candv, nvec)
                tk = plsc.load_gather(taken_v, [csafe])
                avail = jnp.logical_and(tk == 0, candv < nvec)
                f = plsc.all_reduce_ffs(avail)
                chn = candv.at[jnp.minimum(f, s15)].get(
                    mode="promise_in_bounds")
                return jnp.where(f < 16, chn, -1), candv

            chv, candv0 = scan_vreg(jnp.int32(0))
            last_real = candv0.at[s15].get(mode="promise_in_bounds") < nvec
            pred = jnp.any(jnp.logical_and(chv < 0, last_real))

            def tail():
                def body(i, ch2):
                    chn, _ = scan_vreg(i)
                    return jnp.where(ch2 >= 0, ch2, chn)
                return lax.fori_loop(1, (K + 15) // 16, body,
                                     jnp.full((16,), -1, jnp.int32))

            chv = lax.cond(pred, tail, lambda: chv)
            has = chv >= 0
            csafe = jnp.maximum(chv, 0)
            plsc.store_scatter(taken_v, [csafe], ones,
                               mask=jnp.logical_and(lane0, has))
            sv = plsc.load_gather(scores_v, [csafe])
            chosen_v[pl.ds(c * 16, 16)] = chv
            svals_v[pl.ds(c * 16, 16)] = jnp.where(has, sv, 0.0)
            return carry

        negs = jnp.full((16,), -1, jnp.int32)
        fzeros = jnp.zeros((16,), jnp.float32)

        def padb(c, carry):
            chosen_v[pl.ds(c * 16, 16)] = negs
            svals_v[pl.ds(c * 16, 16)] = fzeros
            return carry

        lax.fori_loop(NLBL, ML, padb, 0)
        lax.fori_loop(0, NLBL, lbody, 0)
        pltpu.sync_copy(chosen_v, chosen_hbm)
        pltpu.sync_copy(svals_v, svals_hbm)


def _k3_body(scores_hbm, chosen_hbm, svals_hbm, ranks_hbm,
             scores_v, ch_v, sv_v, rk_v):
    w = _wid()
    pltpu.sync_copy(scores_hbm, scores_v)
    pltpu.sync_copy(chosen_hbm.at[pl.ds(w * LPW * 16, LPW * 16)], ch_v)
    pltpu.sync_copy(svals_hbm.at[pl.ds(w * LPW * 16, LPW * 16)], sv_v)
    lanes = lax.iota(jnp.int32, 16)
    jv = [ch_v[pl.ds(j * 16, 16)] for j in range(LPW)]
    sv = [sv_v[pl.ds(j * 16, 16)] for j in range(LPW)]

    def body(i, accs):
        b = i * 16
        sc = scores_v[pl.ds(b, 16)]
        idxv = lanes + b
        return tuple(
            accs[j] + jnp.logical_or(
                sc > sv[j],
                jnp.logical_and(sc == sv[j], idxv < jv[j])
            ).astype(jnp.int32)
            for j in range(LPW))

    accs = lax.fori_loop(
        0, NV, body, tuple(jnp.zeros((16,), jnp.int32) for _ in range(LPW)))
    lane0 = lanes == 0
    for j in range(LPW + 1):
        if j < LPW:
            rank = jnp.where(jnp.max(jv[j]) >= 0, jnp.sum(accs[j]), BIG)
        else:
            rank = jnp.int32(BIG)
        plsc.store_scatter(rk_v, [jnp.full((16,), j, jnp.int32)],
                           jnp.full((16,), rank, jnp.int32), mask=lane0)
    pltpu.sync_copy(rk_v, ranks_hbm.at[pl.ds(w * 8, 8)])


def _k4_body(ranks_hbm, ap_hbm, rk_v, p_v, out_v):
    w = _wid()

    @pl.when(w == 0)
    def _():
        pltpu.sync_copy(ranks_hbm, rk_v)
        lanes = lax.iota(jnp.int32, 16)
        lane0 = lanes == 0

        neg1 = jnp.full((16,), -1.0, jnp.float32)

        def spi(i, carry):
            p_v[pl.ds(i * 16, 16)] = neg1
            return carry

        lax.fori_loop(0, NSV, spi, 0)

        def pa(c, carry):
            cv = jnp.full((16,), c, jnp.int32)
            rcv = plsc.load_gather(rk_v, [cv])

            def ib(i, acc):
                rv = rk_v[pl.ds(i * 16, 16)]
                return acc + (rv < rcv).astype(jnp.int32)

            acc = lax.fori_loop(0, NSV, ib, jnp.zeros((16,), jnp.int32))
            pos = jnp.sum(acc)
            validv = rcv < BIG
            pvec = (jnp.full((16,), pos, jnp.float32) + 1.0) / \
                   (rcv.astype(jnp.float32) + 1.0)
            posv = jnp.full((16,), pos, jnp.int32)
            plsc.store_scatter(p_v, [posv], pvec,
                               mask=jnp.logical_and(lane0, validv))
            return carry

        lax.fori_loop(0, NSLOT, pa, 0)

        def r0b(i, acc):
            rv = rk_v[pl.ds(i * 16, 16)]
            return jnp.logical_or(acc, rv == 0)

        r0v = lax.fori_loop(0, NSV, r0b, lanes != lanes)
        has_r0 = jnp.any(r0v)

        def pb(t, carry):
            runmax, ap = carry
            i = NSV - 1 - t
            v = p_v[pl.ds(i * 16, 16)]
            suf = lax.rev(plsc.cummax(lax.rev(v, dimensions=(0,))),
                          dimensions=(0,))
            tot = jnp.maximum(suf, runmax)
            ap = ap + jnp.sum(jnp.where(v >= 0.0, tot, 0.0))
            return jnp.full((16,), jnp.max(tot), jnp.float32), ap

        runmax, ap = lax.fori_loop(0, NSV, pb, (neg1, jnp.float32(0.0)))
        ap = ap - jnp.where(has_r0, jnp.max(runmax), 0.0)
        out_v[...] = jnp.full((16,), ap * jnp.float32(1.0 / NLBL),
                              jnp.float32)
        pltpu.sync_copy(out_v, ap_hbm)


@functools.lru_cache(maxsize=1)
def _build():
    mesh = plsc.VectorSubcoreMesh(
        core_axis_name="c", subcore_axis_name="s",
        num_cores=NCORES, num_subcores=NSUB)
    mesh1 = plsc.VectorSubcoreMesh(
        core_axis_name="c", subcore_axis_name="s",
        num_cores=1, num_subcores=1)
    k1 = pl.kernel(
        _k1_body, mesh=mesh,
        compiler_params=pltpu.CompilerParams(needs_layout_passes=False),
        out_type=jax.ShapeDtypeStruct((ML * CAP,), jnp.int32),
        scratch_types=[pltpu.VMEM((N,), jnp.float32),
                       pltpu.VMEM((N,), jnp.float32),
                       pltpu.VMEM((LPW * 16,), jnp.float32),
                       pltpu.VMEM((LPW * 16,), jnp.float32),
                       pltpu.VMEM((LPW * CAP,), jnp.int32)])
    k2 = pl.kernel(
        _k2_body, mesh=mesh1,
        compiler_params=pltpu.CompilerParams(needs_layout_passes=False),
        out_type=[jax.ShapeDtypeStruct((ML * 16,), jnp.int32),
                  jax.ShapeDtypeStruct((ML * 16,), jnp.float32)],
        scratch_types=[pltpu.VMEM((ML * CAP,), jnp.int32),
                       pltpu.VMEM((N,), jnp.float32),
                       pltpu.VMEM((N + 16,), jnp.int32),
                       pltpu.VMEM((ML * 16,), jnp.int32),
                       pltpu.VMEM((ML * 16,), jnp.float32)])
    k3 = pl.kernel(
        _k3_body, mesh=mesh,
        compiler_params=pltpu.CompilerParams(needs_layout_passes=False),
        out_type=jax.ShapeDtypeStruct((NSLOT,), jnp.int32),
        scratch_types=[pltpu.VMEM((N,), jnp.float32),
                       pltpu.VMEM((LPW * 16,), jnp.int32),
                       pltpu.VMEM((LPW * 16,), jnp.float32),
                       pltpu.VMEM((8,), jnp.int32)])
    k4 = pl.kernel(
        _k4_body, mesh=mesh1,
        compiler_params=pltpu.CompilerParams(needs_layout_passes=False),
        out_type=jax.ShapeDtypeStruct((16,), jnp.float32),
        scratch_types=[pltpu.VMEM((NSLOT,), jnp.int32),
                       pltpu.VMEM((NSLOT,), jnp.float32),
                       pltpu.VMEM((16,), jnp.float32)])
    return k1, k2, k3, k4


def kernel(scores, segments, labels):
    smin = segments[:, 0]
    smax = segments[:, 1]
    lmin = jnp.concatenate(
        [labels[:, 0], jnp.full((ML - NLBL,), -2.0, jnp.float32)])
    lmax = jnp.concatenate(
        [labels[:, 1], jnp.full((ML - NLBL,), -1.0, jnp.float32)])
    lmin16 = jnp.broadcast_to(lmin[:, None], (ML, 16)).reshape(ML * 16)
    lmax16 = jnp.broadcast_to(lmax[:, None], (ML, 16)).reshape(ML * 16)

    k1, k2, k3, k4 = _build()
    lists = k1(smin, smax, lmin16, lmax16)
    chosen, svals = k2(lists, scores)
    ranks = k3(scores, chosen, svals)
    ap16 = k4(ranks)
    return ap16[0]


# speedup vs baseline: 2.0485x; 1.0001x over previous
import functools

import jax
import jax.numpy as jnp
from jax import lax
from jax.experimental import pallas as pl
from jax.experimental.pallas import tpu as pltpu
from jax.experimental.pallas import tpu_sc as plsc

N = 20000
NV = N // 16
NLBL = 200
NCORES = 2
NSUB = 16
NW = NCORES * NSUB
LPW = 7
ML = NW * LPW
CAP = 224
K = 200
BIG = 1 << 30
NSLOT = NW * 8
NSV = NSLOT // 16

def _wid():
    return lax.axis_index("s") * NCORES + lax.axis_index("c")


def _k1_body(smin_hbm, smax_hbm, lmin_hbm, lmax_hbm, lists_hbm,
             smin_v, smax_v, lmin_v, lmax_v, list_v):
    w = _wid()
    base_l = w * LPW
    pltpu.sync_copy(smin_hbm, smin_v)
    pltpu.sync_copy(smax_hbm, smax_v)
    pltpu.sync_copy(lmin_hbm.at[pl.ds(base_l * 16, LPW * 16)], lmin_v)
    pltpu.sync_copy(lmax_hbm.at[pl.ds(base_l * 16, LPW * 16)], lmax_v)
    lanes = lax.iota(jnp.int32, 16)

    bigv = jnp.full((16,), BIG, jnp.int32)

    def initb(i, carry):
        list_v[pl.ds(i * 16, 16)] = bigv
        return carry

    lax.fori_loop(0, LPW * CAP // 16, initb, 0)
    bmin = [lmin_v[pl.ds(j * 16, 16)] for j in range(LPW)]
    bmax = [lmax_v[pl.ds(j * 16, 16)] for j in range(LPW)]
    blen = [bmax[j] - bmin[j] for j in range(LPW)]
    kcap = [jnp.full((16,), j * CAP + K - 1, jnp.int32) for j in range(LPW)]

    def body(i, offs):
        b = i * 16
        sm = smin_v[pl.ds(b, 16)]
        sx = smax_v[pl.ds(b, 16)]
        la = sx - sm
        iv = lanes + b
        new_offs = []
        for j in range(LPW):
            raw = jnp.minimum(sx, bmax[j]) - jnp.maximum(sm, bmin[j])
            m = (raw * 3.0) > (la + blen[j])
            off = offs[j]
            mstore = jnp.logical_and(m, off < kcap[j])
            mi = mstore.astype(jnp.int32)
            dest = jnp.minimum(off, kcap[j]) + plsc.cumsum(mi)
            plsc.store_scatter(list_v, [dest], iv, mask=mstore)
            new_offs.append(off + plsc.all_reduce_population_count(m))
        return tuple(new_offs)

    lax.fori_loop(
        0, NV, body,
        tuple(jnp.full((16,), j * CAP - 1, jnp.int32) for j in range(LPW)))
    pltpu.sync_copy(list_v, lists_hbm.at[pl.ds(base_l * CAP, LPW * CAP)])


def _k2_body(lists_hbm, scores_hbm, chosen_hbm, svals_hbm,
             lists_v, scores_v, taken_v, chosen_v, svals_v):
    w = _wid()

    @pl.when(w == 0)
    def _():
        pltpu.sync_copy(lists_hbm, lists_v)
        pltpu.sync_copy(scores_hbm, scores_v)
        lanes = lax.iota(jnp.int32, 16)
        zeros = jnp.zeros((16,), jnp.int32)
        ones = jnp.ones((16,), jnp.int32)
        nvec = jnp.full((16,), N, jnp.int32)

        def zbody(i, carry):
            taken_v[pl.ds(i * 16, 16)] = zeros
            return carry

        lax.fori_loop(0, NV + 1, zbody, 0)

        lane0 = lanes == 0
        s15 = jnp.full((16,), 15, jnp.int32)

        def lbody(c, carry):
            def scan_vreg(i):
                candv = lists_v[pl.ds(c * CAP + i * 16, 16)]
                csafe = jnp.minimum(candv, nvec)
                tk = plsc.load_gather(taken_v, [csafe])
                avail = jnp.logical_and(tk == 0, candv < nvec)
                f = plsc.all_reduce_ffs(avail)
                chn = candv.at[jnp.minimum(f, s15)].get(
                    mode="promise_in_bounds")
                return jnp.where(f < 16, chn, -1), candv

            chv, candv0 = scan_vreg(jnp.int32(0))
            last_real = candv0.at[s15].get(mode="promise_in_bounds") < nvec
            pred = jnp.any(jnp.logical_and(chv < 0, last_real))

            def tail():
                def body(i, ch2):
                    chn, _ = scan_vreg(i)
                    return jnp.where(ch2 >= 0, ch2, chn)
                return lax.fori_loop(1, (K + 15) // 16, body,
                                     jnp.full((16,), -1, jnp.int32))

            chv = lax.cond(pred, tail, lambda: chv)
            has = chv >= 0
            csafe = jnp.maximum(chv, 0)
            plsc.store_scatter(taken_v, [csafe], ones,
                               mask=jnp.logical_and(lane0, has))
            sv = plsc.load_gather(scores_v, [csafe])
            chosen_v[pl.ds(c * 16, 16)] = chv
            svals_v[pl.ds(c * 16, 16)] = jnp.where(has, sv, 0.0)
            return carry

        negs = jnp.full((16,), -1, jnp.int32)
        fzeros = jnp.zeros((16,), jnp.float32)

        def padb(c, carry):
            chosen_v[pl.ds(c * 16, 16)] = negs
            svals_v[pl.ds(c * 16, 16)] = fzeros
            return carry

        lax.fori_loop(NLBL, ML, padb, 0)
        lax.fori_loop(0, NLBL, lbody, 0)
        pltpu.sync_copy(chosen_v, chosen_hbm)
        pltpu.sync_copy(svals_v, svals_hbm)


def _k3_body(scores_hbm, chosen_hbm, svals_hbm, ranks_hbm,
             scores_v, ch_v, sv_v, rk_v):
    w = _wid()
    pltpu.sync_copy(scores_hbm, scores_v)
    pltpu.sync_copy(chosen_hbm.at[pl.ds(w * LPW * 16, LPW * 16)], ch_v)
    pltpu.sync_copy(svals_hbm.at[pl.ds(w * LPW * 16, LPW * 16)], sv_v)
    lanes = lax.iota(jnp.int32, 16)
    jv = [ch_v[pl.ds(j * 16, 16)] for j in range(LPW)]
    sv = [sv_v[pl.ds(j * 16, 16)] for j in range(LPW)]

    def body(i, accs):
        b = i * 16
        sc = scores_v[pl.ds(b, 16)]
        idxv = lanes + b
        return tuple(
            accs[j] + jnp.logical_or(
                sc > sv[j],
                jnp.logical_and(sc == sv[j], idxv < jv[j])
            ).astype(jnp.int32)
            for j in range(LPW))

    accs = lax.fori_loop(
        0, NV, body, tuple(jnp.zeros((16,), jnp.int32) for _ in range(LPW)))
    lane0 = lanes == 0
    for j in range(LPW + 1):
        if j < LPW:
            rank = jnp.where(jnp.max(jv[j]) >= 0, jnp.sum(accs[j]), BIG)
        else:
            rank = jnp.int32(BIG)
        plsc.store_scatter(rk_v, [jnp.full((16,), j, jnp.int32)],
                           jnp.full((16,), rank, jnp.int32), mask=lane0)
    pltpu.sync_copy(rk_v, ranks_hbm.at[pl.ds(w * 8, 8)])


def _k4_body(ranks_hbm, ap_hbm, rk_v, p_v, out_v):
    w = _wid()

    @pl.when(w == 0)
    def _():
        pltpu.sync_copy(ranks_hbm, rk_v)
        lanes = lax.iota(jnp.int32, 16)
        lane0 = lanes == 0

        neg1 = jnp.full((16,), -1.0, jnp.float32)

        def spi(i, carry):
            p_v[pl.ds(i * 16, 16)] = neg1
            return carry

        lax.fori_loop(0, NSV, spi, 0)

        def pa(c, carry):
            cv = jnp.full((16,), c, jnp.int32)
            rcv = plsc.load_gather(rk_v, [cv])

            def ib(i, acc):
                rv = rk_v[pl.ds(i * 16, 16)]
                return acc + (rv < rcv).astype(jnp.int32)

            acc = lax.fori_loop(0, NSV, ib, jnp.zeros((16,), jnp.int32))
            pos = jnp.sum(acc)
            validv = rcv < BIG
            pvec = (jnp.full((16,), pos, jnp.float32) + 1.0) / \
                   (rcv.astype(jnp.float32) + 1.0)
            posv = jnp.full((16,), pos, jnp.int32)
            plsc.store_scatter(p_v, [posv], pvec,
                               mask=jnp.logical_and(lane0, validv))
            return carry

        lax.fori_loop(0, NSLOT, pa, 0)

        def r0b(i, acc):
            rv = rk_v[pl.ds(i * 16, 16)]
            return jnp.logical_or(acc, rv == 0)

        r0v = lax.fori_loop(0, NSV, r0b, lanes != lanes)
        has_r0 = jnp.any(r0v)

        def pb(t, carry):
            runmax, ap = carry
            i = NSV - 1 - t
            v = p_v[pl.ds(i * 16, 16)]
            suf = lax.rev(plsc.cummax(lax.rev(v, dimensions=(0,))),
                          dimensions=(0,))
            tot = jnp.maximum(suf, runmax)
            ap = ap + jnp.sum(jnp.where(v >= 0.0, tot, 0.0))
            return jnp.full((16,), jnp.max(tot), jnp.float32), ap

        runmax, ap = lax.fori_loop(0, NSV, pb, (neg1, jnp.float32(0.0)))
        ap = ap - jnp.where(has_r0, jnp.max(runmax), 0.0)
        out_v[...] = jnp.full((16,), ap * jnp.float32(1.0 / NLBL),
                              jnp.float32)
        pltpu.sync_copy(out_v, ap_hbm)


@functools.lru_cache(maxsize=1)
def _build():
    mesh = plsc.VectorSubcoreMesh(
        core_axis_name="c", subcore_axis_name="s",
        num_cores=NCORES, num_subcores=NSUB)
    mesh1 = plsc.VectorSubcoreMesh(
        core_axis_name="c", subcore_axis_name="s",
        num_cores=1, num_subcores=1)
    k1 = pl.kernel(
        _k1_body, mesh=mesh,
        compiler_params=pltpu.CompilerParams(needs_layout_passes=False),
        out_type=jax.ShapeDtypeStruct((ML * CAP,), jnp.int32),
        scratch_types=[pltpu.VMEM((N,), jnp.float32),
                       pltpu.VMEM((N,), jnp.float32),
                       pltpu.VMEM((LPW * 16,), jnp.float32),
                       pltpu.VMEM((LPW * 16,), jnp.float32),
                       pltpu.VMEM((LPW * CAP,), jnp.int32)])
    k2 = pl.kernel(
        _k2_body, mesh=mesh1,
        compiler_params=pltpu.CompilerParams(needs_layout_passes=False),
        out_type=[jax.ShapeDtypeStruct((ML * 16,), jnp.int32),
                  jax.ShapeDtypeStruct((ML * 16,), jnp.float32)],
        scratch_types=[pltpu.VMEM((ML * CAP,), jnp.int32),
                       pltpu.VMEM((N,), jnp.float32),
                       pltpu.VMEM((N + 16,), jnp.int32),
                       pltpu.VMEM((ML * 16,), jnp.int32),
                       pltpu.VMEM((ML * 16,), jnp.float32)])
    k3 = pl.kernel(
        _k3_body, mesh=mesh,
        compiler_params=pltpu.CompilerParams(needs_layout_passes=False),
        out_type=jax.ShapeDtypeStruct((NSLOT,), jnp.int32),
        scratch_types=[pltpu.VMEM((N,), jnp.float32),
                       pltpu.VMEM((LPW * 16,), jnp.int32),
                       pltpu.VMEM((LPW * 16,), jnp.float32),
                       pltpu.VMEM((8,), jnp.int32)])
    k4 = pl.kernel(
        _k4_body, mesh=mesh1,
        compiler_params=pltpu.CompilerParams(needs_layout_passes=False),
        out_type=jax.ShapeDtypeStruct((16,), jnp.float32),
        scratch_types=[pltpu.VMEM((NSLOT,), jnp.int32),
                       pltpu.VMEM((NSLOT,), jnp.float32),
                       pltpu.VMEM((16,), jnp.float32)])
    return k1, k2, k3, k4


def kernel(scores, segments, labels):
    smin = segments[:, 0]
    smax = segments[:, 1]
    lmin = jnp.concatenate(
        [labels[:, 0], jnp.full((ML - NLBL,), -2.0, jnp.float32)])
    lmax = jnp.concatenate(
        [labels[:, 1], jnp.full((ML - NLBL,), -1.0, jnp.float32)])
    lmin16 = jnp.broadcast_to(lmin[:, None], (ML, 16)).reshape(ML * 16)
    lmax16 = jnp.broadcast_to(lmax[:, None], (ML, 16)).reshape(ML * 16)

    k1, k2, k3, k4 = _build()
    lists = k1(smin, smax, lmin16, lmax16)
    chosen, svals = k2(lists, scores)
    ranks = k3(scores, chosen, svals)
    ap16 = k4(ranks)
    return ap16[0]
